# baseline (speedup 1.0000x reference)
import functools

import jax
import jax.numpy as jnp
from jax import lax
from jax.experimental import pallas as pl
from jax.experimental.pallas import tpu as pltpu
from jax.experimental.pallas import tpu_sc as plsc

NUM_EXPERTS = 8
IN_DIM = 1024
HIDDEN_DIM = 2736
SEQ = 2048
NASSIGN = 2 * SEQ

BT = 512
NSLOTS = NASSIGN + NUM_EXPERTS * BT
NBLOCKS = NSLOTS // BT
NBPAD = 32
DT = 256
ND = IN_DIM // DT
NH = 3
HT = HIDDEN_DIM // NH
NPH = ND + NH

NC, NS, L = 2, 16, 16
NW = NC * NS


def _router_kernel(x_ref, c_ref, sel_ref, wts_ref):
    xs = x_ref[...]
    cen = c_ref[...]
    xc = jnp.dot(xs, cen.T, preferred_element_type=jnp.float32,
                 precision=jax.lax.Precision.HIGHEST)
    xn = jnp.sum(xs * xs, axis=1, keepdims=True)
    cn = jnp.sum(cen * cen, axis=1)[None, :]
    dist = jnp.sqrt(jnp.maximum(xn - 2.0 * xc + cn, 0.0))
    m1 = jnp.max(dist, axis=1, keepdims=True)
    idx = jax.lax.broadcasted_iota(jnp.int32, dist.shape, 1)
    a1 = jnp.min(jnp.where(dist == m1, idx, NUM_EXPERTS), axis=1,
                 keepdims=True)
    masked = jnp.where(idx == a1, -jnp.inf, dist)
    m2 = jnp.max(masked, axis=1, keepdims=True)
    a2 = jnp.min(jnp.where(masked == m2, idx, NUM_EXPERTS), axis=1,
                 keepdims=True)
    z2 = jnp.exp(m2 - m1)
    w1 = 1.0 / (1.0 + z2)
    w2 = z2 / (1.0 + z2)
    sel_ref[...] = jnp.concatenate([a1, a2], axis=1)
    wts_ref[...] = jnp.concatenate([w1, w2], axis=1)


def _dispatch_body(sel_hbm, wts_hbm, srctok_hbm, gate_hbm, pos_hbm,
                   btab_hbm, sel_v, wts_v, srctok_v, gate_v, pos_v,
                   btab_v, cur_s):
    wid = lax.axis_index("s") * NC + lax.axis_index("c")

    @pl.when(wid == 0)
    def _():
        pltpu.sync_copy(sel_hbm, sel_v)
        pltpu.sync_copy(wts_hbm, wts_v)
        zi = jnp.zeros((L,), jnp.int32)
        zf = jnp.zeros((L,), jnp.float32)

        def _zero(i, _):
            srctok_v[pl.ds(i * L, L)] = zi
            gate_v[pl.ds(i * L, L)] = zf
            return _
        lax.fori_loop(0, NSLOTS // L, _zero, None)

        def _count(i, cnt):
            v = sel_v[pl.ds(i * L, L)]
            lanes = lax.iota(jnp.int32, L)
            for e in range(NUM_EXPERTS):
                ce = jnp.sum(jnp.where(v == e, 1, 0))
                cnt = cnt + jnp.where(lanes == e, ce, 0)
            return cnt
        cnt = lax.fori_loop(0, NASSIGN // L, _count, jnp.zeros((L,), jnp.int32))

        nblk = (cnt + (BT - 1)) // BT
        csum = plsc.cumsum(nblk)
        first_blk = csum - nblk
        seg_start = first_blk * BT
        total = jnp.sum(nblk)

        for e in range(NUM_EXPERTS):
            cur_s[e] = seg_start[e]

        ce_list = [csum[e] for e in range(NUM_EXPERTS)]
        last_e = jnp.int32(0)
        for ce in ce_list:
            last_e = last_e + jnp.where(ce <= total - 1, 1, 0)
        for c in range(NBPAD // L):
            bvec = lax.iota(jnp.int32, L) + c * L
            bexp = jnp.zeros((L,), jnp.int32)
            for ce in ce_list:
                bexp = bexp + jnp.where(bvec >= ce, 1, 0)
            valid = bvec < total
            bexp = jnp.where(valid, bexp, last_e)
            bfetch = jnp.where(valid, bvec, total - 1)
            btab_v[pl.ds(c * L, L)] = bexp
            btab_v[pl.ds(NBPAD + c * L, L)] = bfetch
            btab_v[pl.ds(2 * NBPAD + c * L, L)] = jnp.where(valid, 1, 0)

        def _scatter(i, _):
            v = sel_v[pl.ds(i * L, L)]
            w = wts_v[pl.ds(i * L, L)]
            tok = (lax.iota(jnp.int32, L) + i * L) // 2
            posv = jnp.zeros((L,), jnp.int32)
            for e in range(NUM_EXPERTS):
                m = v == e
                mi = jnp.where(m, 1, 0)
                rank = plsc.cumsum(mi) - 1
                base = cur_s[e]
                posv = jnp.where(m, base + rank, posv)
                cur_s[e] = base + jnp.sum(mi)
            plsc.store_scatter(srctok_v, [posv], tok)
            plsc.store_scatter(gate_v, [posv], w)
            pos_v[pl.ds(i * L, L)] = posv
            return _
        lax.fori_loop(0, NASSIGN // L, _scatter, None)

        pltpu.sync_copy(srctok_v, srctok_hbm)
        pltpu.sync_copy(gate_v, gate_hbm)
        pltpu.sync_copy(pos_v, pos_hbm)
        pltpu.sync_copy(btab_v, btab_hbm)


def _dgt(a, b):
    return jax.lax.dot_general(a, b, (((0,), (0,)), ((), ())),
                               preferred_element_type=jnp.float32)


def _ffn_kernel(be_ref, bv_ref, bf_ref, x_ref, tokr_ref, wg_ref, bg_ref,
                w1_ref, b1_ref, w2_ref, b2_ref, gater_ref, out_ref,
                xs_scr, g_acc, u_acc, yt_acc):
    b = pl.program_id(0)
    ph = pl.program_id(1)

    @pl.when(bv_ref[b] == 1)
    def _body():
        @pl.when(ph == 0)
        def _gather():
            tok = tokr_ref[0]
            row = jax.lax.broadcasted_iota(jnp.int32, (SEQ, BT), 0)
            oht = jnp.where(row == tok, 1.0, 0.0)
            xs_scr[...] = _dgt(x_ref[...], oht)

        @pl.when(ph < ND)
        def _accum():
            xsd = xs_scr[pl.ds(ph * DT, DT), :]
            g = _dgt(wg_ref[0], xsd)
            u = _dgt(w1_ref[0], xsd)
            for jj in range(NH):
                gj = g[jj * HT:(jj + 1) * HT, :]
                uj = u[jj * HT:(jj + 1) * HT, :]

                @pl.when(ph == 0)
                def _(jj=jj, gj=gj, uj=uj):
                    g_acc[jj] = gj + bg_ref[0, pl.ds(jj * HT, HT), :]
                    u_acc[jj] = uj + b1_ref[0, pl.ds(jj * HT, HT), :]

                @pl.when(ph > 0)
                def _(jj=jj, gj=gj, uj=uj):
                    g_acc[jj] += gj
                    u_acc[jj] += uj

        @pl.when(ph >= ND)
        def _w2():
            j = ph - ND
            gate = gater_ref[0]
            gt = g_acc[j]
            ut = u_acc[j]
            ht = (gate * (gt * jax.nn.sigmoid(gt))) * ut
            yt = _dgt(w2_ref[0], ht)

            @pl.when(j == 0)
            def _():
                yt_acc[...] = yt + b2_ref[0] * gate

            @pl.when(j > 0)
            def _():
                yt_acc[...] += yt

            @pl.when(j == NH - 1)
            def _():
                out_ref[...] = yt_acc[...].T


def _ffn_in_specs():
    def _dclamp(p, bv_b):
        return jnp.where(bv_b == 1, jnp.minimum(p, ND - 1), ND - 1)

    return [
        pl.BlockSpec((SEQ, IN_DIM), lambda b, p, be, bv, bf: (0, 0)),
        pl.BlockSpec((1, 1, BT), lambda b, p, be, bv, bf: (bf[b], 0, 0)),
        pl.BlockSpec((1, DT, HIDDEN_DIM),
                     lambda b, p, be, bv, bf: (be[b], _dclamp(p, bv[b]), 0)),
        pl.BlockSpec((1, HIDDEN_DIM, 1),
                     lambda b, p, be, bv, bf: (be[b], 0, 0)),
        pl.BlockSpec((1, DT, HIDDEN_DIM),
                     lambda b, p, be, bv, bf: (be[b], _dclamp(p, bv[b]), 0)),
        pl.BlockSpec((1, HIDDEN_DIM, 1),
                     lambda b, p, be, bv, bf: (be[b], 0, 0)),
        pl.BlockSpec((1, HT, IN_DIM),
                     lambda b, p, be, bv, bf:
                     (be[b], jnp.where(bv[b] == 1,
                                       jnp.clip(p - ND, 0, NH - 1),
                                       NH - 1), 0)),
        pl.BlockSpec((1, IN_DIM, 1),
                     lambda b, p, be, bv, bf: (be[b], 0, 0)),
        pl.BlockSpec((1, 1, BT), lambda b, p, be, bv, bf: (bf[b], 0, 0)),
    ]


def _ffn_out_spec():
    return pl.BlockSpec((BT, IN_DIM), lambda b, p, be, bv, bf: (bf[b], 0))


def _ffn_scratch():
    return [pltpu.VMEM((IN_DIM, BT), jnp.float32),
            pltpu.VMEM((NH, HT, BT), jnp.float32),
            pltpu.VMEM((NH, HT, BT), jnp.float32),
            pltpu.VMEM((IN_DIM, BT), jnp.float32)]


def _combine_body(y_hbm, pos_hbm, out_hbm, idx_v, buf_v, obuf_v, sem):
    wid = lax.axis_index("s") * NC + lax.axis_index("c")
    tok_per_w = SEQ // NW
    base_t = wid * tok_per_w
    pltpu.sync_copy(pos_hbm.at[pl.ds(base_t * 2, tok_per_w * 2)], idx_v)
    chunk = 32
    for c in range(tok_per_w // chunk):
        pltpu.async_copy(y_hbm.at[idx_v.at[pl.ds(c * chunk * 2, chunk * 2)]],
                         buf_v, sem).wait()

        def _comb(i, _):
            for j in range(IN_DIM // L):
                s = pl.ds(j * L, L)
                obuf_v[i, s] = buf_v[2 * i, s] + buf_v[2 * i + 1, s]
            return _
        lax.fori_loop(0, chunk, _comb, None)
        pltpu.sync_copy(obuf_v,
                        out_hbm.at[pl.ds(base_t + c * chunk, chunk)])


@jax.jit
def _moe_forward(xs, centroid, Wg, bg, W1, b1, W2, b2):
    _sc_mesh = plsc.VectorSubcoreMesh(core_axis_name="c", subcore_axis_name="s")
    sel, wts = pl.pallas_call(
        _router_kernel,
        out_shape=[jax.ShapeDtypeStruct((SEQ, 2), jnp.int32),
                   jax.ShapeDtypeStruct((SEQ, 2), jnp.float32)],
    )(xs, centroid)

    dispatch = pl.kernel(
        _dispatch_body, mesh=_sc_mesh,
        out_type=[jax.ShapeDtypeStruct((NSLOTS,), jnp.int32),
                  jax.ShapeDtypeStruct((NSLOTS,), jnp.float32),
                  jax.ShapeDtypeStruct((NASSIGN,), jnp.int32),
                  jax.ShapeDtypeStruct((3 * NBPAD,), jnp.int32)],
        scratch_types=[pltpu.VMEM((NASSIGN,), jnp.int32),
                       pltpu.VMEM((NASSIGN,), jnp.float32),
                       pltpu.VMEM((NSLOTS,), jnp.int32),
                       pltpu.VMEM((NSLOTS,), jnp.float32),
                       pltpu.VMEM((NASSIGN,), jnp.int32),
                       pltpu.VMEM((3 * NBPAD,), jnp.int32),
                       pltpu.SMEM((NUM_EXPERTS,), jnp.int32)],
        compiler_params=pltpu.CompilerParams(needs_layout_passes=False),
    )
    srctok, slot_gate, pos, btab = dispatch(sel.reshape(NASSIGN),
                                            wts.reshape(NASSIGN))

    btab32 = btab.reshape(3, NBPAD)
    bexp, bfetch, bval = btab32[0], btab32[1], btab32[2]

    grid_spec = pltpu.PrefetchScalarGridSpec(
        num_scalar_prefetch=3,
        grid=(NBLOCKS, NPH),
        in_specs=_ffn_in_specs(),
        out_specs=_ffn_out_spec(),
        scratch_shapes=_ffn_scratch(),
    )
    y_sorted = pl.pallas_call(
        _ffn_kernel,
        grid_spec=grid_spec,
        out_shape=jax.ShapeDtypeStruct((NSLOTS, IN_DIM), jnp.float32),
        compiler_params=pltpu.CompilerParams(
            fuse_transposed_lhs_in_matmul=True,
            vmem_limit_bytes=64 * 1024 * 1024),
    )(bexp, bval, bfetch, xs, srctok.reshape(NBLOCKS, 1, BT), Wg,
      bg.reshape(NUM_EXPERTS, HIDDEN_DIM, 1), W1,
      b1.reshape(NUM_EXPERTS, HIDDEN_DIM, 1), W2,
      b2.reshape(NUM_EXPERTS, IN_DIM, 1), slot_gate.reshape(NBLOCKS, 1, BT))

    combine = pl.kernel(
        _combine_body, mesh=_sc_mesh,
        out_type=[jax.ShapeDtypeStruct((SEQ, IN_DIM), jnp.float32)],
        scratch_types=[pltpu.VMEM((2 * SEQ // NW,), jnp.int32),
                       pltpu.VMEM((64, IN_DIM), jnp.float32),
                       pltpu.VMEM((32, IN_DIM), jnp.float32),
                       pltpu.SemaphoreType.DMA],
        compiler_params=pltpu.CompilerParams(needs_layout_passes=False),
    )
    (out,) = combine(y_sorted, pos)
    return out


def kernel(x, centroid, Wg, bg, W1, b1, W2, b2):
    xs = x.reshape(-1, IN_DIM)
    out = _moe_forward(xs, centroid, Wg, bg, W1, b1, W2, b2)
    return out.reshape(x.shape)


# speedup vs baseline: 1.1540x; 1.1540x over previous
import functools

import jax
import jax.numpy as jnp
from jax import lax
from jax.experimental import pallas as pl
from jax.experimental.pallas import tpu as pltpu
from jax.experimental.pallas import tpu_sc as plsc

NUM_EXPERTS = 8
IN_DIM = 1024
HIDDEN_DIM = 2736
SEQ = 2048
NASSIGN = 2 * SEQ

BT = 512
NSLOTS = NASSIGN + NUM_EXPERTS * BT
NBLOCKS = NSLOTS // BT
NBPAD = 32
DT = 512
ND = IN_DIM // DT
NH = 3
HT = HIDDEN_DIM // NH
NPH = ND + NH

NC, NS, L = 2, 16, 16
NW = NC * NS


def _router_kernel(x_ref, c_ref, sel_ref, wts_ref, xbf_ref):
    xs = x_ref[...]
    cen = c_ref[...]
    xc = jnp.dot(xs, cen.T, preferred_element_type=jnp.float32,
                 precision=jax.lax.Precision.HIGHEST)
    xn = jnp.sum(xs * xs, axis=1, keepdims=True)
    cn = jnp.sum(cen * cen, axis=1)[None, :]
    dist = jnp.sqrt(jnp.maximum(xn - 2.0 * xc + cn, 0.0))
    m1 = jnp.max(dist, axis=1, keepdims=True)
    idx = jax.lax.broadcasted_iota(jnp.int32, dist.shape, 1)
    a1 = jnp.min(jnp.where(dist == m1, idx, NUM_EXPERTS), axis=1,
                 keepdims=True)
    masked = jnp.where(idx == a1, -jnp.inf, dist)
    m2 = jnp.max(masked, axis=1, keepdims=True)
    a2 = jnp.min(jnp.where(masked == m2, idx, NUM_EXPERTS), axis=1,
                 keepdims=True)
    z2 = jnp.exp(m2 - m1)
    w1 = 1.0 / (1.0 + z2)
    w2 = z2 / (1.0 + z2)
    sel_ref[...] = jnp.concatenate([a1, a2], axis=1)
    wts_ref[...] = jnp.concatenate([w1, w2], axis=1)
    xbf_ref[...] = xs.astype(jnp.bfloat16)


def _dispatch_body(sel_hbm, wts_hbm, srctok_hbm, gate_hbm, pos_hbm,
                   btab_hbm, sel_v, wts_v, srctok_v, gate_v, pos_v,
                   btab_v, cur_s):
    wid = lax.axis_index("s") * NC + lax.axis_index("c")

    @pl.when(wid == 0)
    def _():
        pltpu.sync_copy(sel_hbm, sel_v)
        pltpu.sync_copy(wts_hbm, wts_v)
        zi = jnp.zeros((L,), jnp.int32)
        zf = jnp.zeros((L,), jnp.float32)

        def _zero(i, _):
            srctok_v[pl.ds(i * L, L)] = zi
            gate_v[pl.ds(i * L, L)] = zf
            return _
        lax.fori_loop(0, NSLOTS // L, _zero, None)

        def _count(i, cnt):
            v = sel_v[pl.ds(i * L, L)]
            lanes = lax.iota(jnp.int32, L)
            for e in range(NUM_EXPERTS):
                ce = jnp.sum(jnp.where(v == e, 1, 0))
                cnt = cnt + jnp.where(lanes == e, ce, 0)
            return cnt
        cnt = lax.fori_loop(0, NASSIGN // L, _count, jnp.zeros((L,), jnp.int32))

        nblk = (cnt + (BT - 1)) // BT
        csum = plsc.cumsum(nblk)
        first_blk = csum - nblk
        seg_start = first_blk * BT
        total = jnp.sum(nblk)

        for e in range(NUM_EXPERTS):
            cur_s[e] = seg_start[e]

        ce_list = [csum[e] for e in range(NUM_EXPERTS)]
        last_e = jnp.int32(0)
        for ce in ce_list:
            last_e = last_e + jnp.where(ce <= total - 1, 1, 0)
        for c in range(NBPAD // L):
            bvec = lax.iota(jnp.int32, L) + c * L
            bexp = jnp.zeros((L,), jnp.int32)
            for ce in ce_list:
                bexp = bexp + jnp.where(bvec >= ce, 1, 0)
            valid = bvec < total
            bexp = jnp.where(valid, bexp, last_e)
            bfetch = jnp.where(valid, bvec, total - 1)
            btab_v[pl.ds(c * L, L)] = bexp
            btab_v[pl.ds(NBPAD + c * L, L)] = bfetch
            btab_v[pl.ds(2 * NBPAD + c * L, L)] = jnp.where(valid, 1, 0)

        def _scatter(i, _):
            v = sel_v[pl.ds(i * L, L)]
            w = wts_v[pl.ds(i * L, L)]
            tok = (lax.iota(jnp.int32, L) + i * L) // 2
            posv = jnp.zeros((L,), jnp.int32)
            for e in range(NUM_EXPERTS):
                m = v == e
                mi = jnp.where(m, 1, 0)
                rank = plsc.cumsum(mi) - 1
                base = cur_s[e]
                posv = jnp.where(m, base + rank, posv)
                cur_s[e] = base + jnp.sum(mi)
            plsc.store_scatter(srctok_v, [posv], tok)
            plsc.store_scatter(gate_v, [posv], w)
            pos_v[pl.ds(i * L, L)] = posv
            return _
        lax.fori_loop(0, NASSIGN // L, _scatter, None)

        pltpu.sync_copy(srctok_v, srctok_hbm)
        pltpu.sync_copy(gate_v, gate_hbm)
        pltpu.sync_copy(pos_v, pos_hbm)
        pltpu.sync_copy(btab_v, btab_hbm)


def _ffn_kernel(be_ref, bv_ref, bf_ref, x_ref, tok_ref, wg_ref, bg_ref,
                w1_ref, b1_ref, w2_ref, b2_ref, gate_ref, out_ref,
                g_acc, u_acc, xs_scr):
    b = pl.program_id(0)
    ph = pl.program_id(1)

    @pl.when(bv_ref[b] == 1)
    def _body():
        @pl.when(ph == 0)
        def _gather():
            tok = tok_ref[...]
            col = jax.lax.broadcasted_iota(jnp.int32, (BT, SEQ), 1)
            onehot = jnp.where(col == tok, 1.0, 0.0).astype(jnp.bfloat16)
            xs_scr[...] = jnp.dot(onehot, x_ref[...],
                                  preferred_element_type=jnp.float32)

        @pl.when(ph < ND)
        def _accum():
            xs = xs_scr[:, pl.ds(ph * DT, DT)]
            g = jnp.dot(xs, wg_ref[0], preferred_element_type=jnp.float32)
            u = jnp.dot(xs, w1_ref[0], preferred_element_type=jnp.float32)
            for jj in range(NH):
                gj = g[:, jj * HT:(jj + 1) * HT]
                uj = u[:, jj * HT:(jj + 1) * HT]

                @pl.when(ph == 0)
                def _(jj=jj, gj=gj, uj=uj):
                    g_acc[jj] = gj + bg_ref[0, 0][None, jj * HT:(jj + 1) * HT]
                    u_acc[jj] = uj + b1_ref[0, 0][None, jj * HT:(jj + 1) * HT]

                @pl.when(ph > 0)
                def _(jj=jj, gj=gj, uj=uj):
                    g_acc[jj] += gj
                    u_acc[jj] += uj

        @pl.when(ph >= ND)
        def _w2():
            j = ph - ND
            gate = gate_ref[...]
            g = g_acc[j]
            u = u_acc[j]
            hmid = (gate * (g * jax.nn.sigmoid(g))) * u
            y = jnp.dot(hmid, w2_ref[0], preferred_element_type=jnp.float32)

            @pl.when(j == 0)
            def _():
                out_ref[...] = y + gate * b2_ref[0, 0][None, :]

            @pl.when(j > 0)
            def _():
                out_ref[...] += y


def _ffn_in_specs():
    def _dclamp(p, bv_b):
        return jnp.where(bv_b == 1, jnp.minimum(p, ND - 1), ND - 1)

    return [
        pl.BlockSpec((SEQ, IN_DIM), lambda b, p, be, bv, bf: (0, 0)),
        pl.BlockSpec((BT, 1), lambda b, p, be, bv, bf: (bf[b], 0)),
        pl.BlockSpec((1, DT, HIDDEN_DIM),
                     lambda b, p, be, bv, bf: (be[b], _dclamp(p, bv[b]), 0)),
        pl.BlockSpec((1, 1, HIDDEN_DIM),
                     lambda b, p, be, bv, bf: (be[b], 0, 0)),
        pl.BlockSpec((1, DT, HIDDEN_DIM),
                     lambda b, p, be, bv, bf: (be[b], _dclamp(p, bv[b]), 0)),
        pl.BlockSpec((1, 1, HIDDEN_DIM),
                     lambda b, p, be, bv, bf: (be[b], 0, 0)),
        pl.BlockSpec((1, HT, IN_DIM),
                     lambda b, p, be, bv, bf:
                     (be[b], jnp.where(bv[b] == 1,
                                       jnp.clip(p - ND, 0, NH - 1),
                                       NH - 1), 0)),
        pl.BlockSpec((1, 1, IN_DIM),
                     lambda b, p, be, bv, bf: (be[b], 0, 0)),
        pl.BlockSpec((BT, 1), lambda b, p, be, bv, bf: (bf[b], 0)),
    ]


def _ffn_out_spec():
    return pl.BlockSpec((BT, IN_DIM), lambda b, p, be, bv, bf: (bf[b], 0))


def _ffn_scratch():
    return [pltpu.VMEM((NH, BT, HT), jnp.float32),
            pltpu.VMEM((NH, BT, HT), jnp.float32),
            pltpu.VMEM((BT, IN_DIM), jnp.float32)]


def _combine_body(y_hbm, pos_hbm, out_hbm, idx_v, buf_v, obuf_v, sem):
    wid = lax.axis_index("s") * NC + lax.axis_index("c")
    tok_per_w = SEQ // NW
    base_t = wid * tok_per_w
    pltpu.sync_copy(pos_hbm.at[pl.ds(base_t * 2, tok_per_w * 2)], idx_v)
    chunk = 32
    for c in range(tok_per_w // chunk):
        pltpu.async_copy(y_hbm.at[idx_v.at[pl.ds(c * chunk * 2, chunk * 2)]],
                         buf_v, sem).wait()

        def _comb(i, _):
            for j in range(IN_DIM // L):
                s = pl.ds(j * L, L)
                obuf_v[i, s] = buf_v[2 * i, s] + buf_v[2 * i + 1, s]
            return _
        lax.fori_loop(0, chunk, _comb, None)
        pltpu.sync_copy(obuf_v,
                        out_hbm.at[pl.ds(base_t + c * chunk, chunk)])


@jax.jit
def _moe_forward(xs, centroid, Wg, bg, W1, b1, W2, b2):
    _sc_mesh = plsc.VectorSubcoreMesh(core_axis_name="c", subcore_axis_name="s")
    sel, wts, x_bf = pl.pallas_call(
        _router_kernel,
        out_shape=[jax.ShapeDtypeStruct((SEQ, 2), jnp.int32),
                   jax.ShapeDtypeStruct((SEQ, 2), jnp.float32),
                   jax.ShapeDtypeStruct((SEQ, IN_DIM), jnp.bfloat16)],
    )(xs, centroid)

    dispatch = pl.kernel(
        _dispatch_body, mesh=_sc_mesh,
        out_type=[jax.ShapeDtypeStruct((NSLOTS,), jnp.int32),
                  jax.ShapeDtypeStruct((NSLOTS,), jnp.float32),
                  jax.ShapeDtypeStruct((NASSIGN,), jnp.int32),
                  jax.ShapeDtypeStruct((3 * NBPAD,), jnp.int32)],
        scratch_types=[pltpu.VMEM((NASSIGN,), jnp.int32),
                       pltpu.VMEM((NASSIGN,), jnp.float32),
                       pltpu.VMEM((NSLOTS,), jnp.int32),
                       pltpu.VMEM((NSLOTS,), jnp.float32),
                       pltpu.VMEM((NASSIGN,), jnp.int32),
                       pltpu.VMEM((3 * NBPAD,), jnp.int32),
                       pltpu.SMEM((NUM_EXPERTS,), jnp.int32)],
        compiler_params=pltpu.CompilerParams(needs_layout_passes=False),
    )
    srctok, slot_gate, pos, btab = dispatch(sel.reshape(NASSIGN),
                                            wts.reshape(NASSIGN))

    btab32 = btab.reshape(3, NBPAD)
    bexp, bfetch, bval = btab32[0], btab32[1], btab32[2]

    grid_spec = pltpu.PrefetchScalarGridSpec(
        num_scalar_prefetch=3,
        grid=(NBLOCKS, NPH),
        in_specs=_ffn_in_specs(),
        out_specs=_ffn_out_spec(),
        scratch_shapes=_ffn_scratch(),
    )
    y_sorted = pl.pallas_call(
        _ffn_kernel,
        grid_spec=grid_spec,
        out_shape=jax.ShapeDtypeStruct((NSLOTS, IN_DIM), jnp.float32),
        compiler_params=pltpu.CompilerParams(
            vmem_limit_bytes=64 * 1024 * 1024),
    )(bexp, bval, bfetch, x_bf, srctok.reshape(NSLOTS, 1), Wg,
      bg.reshape(NUM_EXPERTS, 1, HIDDEN_DIM), W1,
      b1.reshape(NUM_EXPERTS, 1, HIDDEN_DIM), W2,
      b2.reshape(NUM_EXPERTS, 1, IN_DIM), slot_gate.reshape(NSLOTS, 1))

    combine = pl.kernel(
        _combine_body, mesh=_sc_mesh,
        out_type=[jax.ShapeDtypeStruct((SEQ, IN_DIM), jnp.float32)],
        scratch_types=[pltpu.VMEM((2 * SEQ // NW,), jnp.int32),
                       pltpu.VMEM((64, IN_DIM), jnp.float32),
                       pltpu.VMEM((32, IN_DIM), jnp.float32),
                       pltpu.SemaphoreType.DMA],
        compiler_params=pltpu.CompilerParams(needs_layout_passes=False),
    )
    (out,) = combine(y_sorted, pos)
    return out


def kernel(x, centroid, Wg, bg, W1, b1, W2, b2):
    xs = x.reshape(-1, IN_DIM)
    out = _moe_forward(xs, centroid, Wg, bg, W1, b1, W2, b2)
    return out.reshape(x.shape)


# speedup vs baseline: 1.1561x; 1.0018x over previous
import jax
import jax.numpy as jnp
from jax import lax
from jax.experimental import pallas as pl
from jax.experimental.pallas import tpu as pltpu
from jax.experimental.pallas import tpu_sc as plsc

NUM_EXPERTS = 8
IN_DIM = 1024
HIDDEN_DIM = 2736
SEQ = 2048
NASSIGN = 2 * SEQ

BT = 512
NSLOTS = NASSIGN + NUM_EXPERTS * BT
NBLOCKS = NSLOTS // BT
NBPAD = 32
DT = 512
ND = IN_DIM // DT
NH = 3
HT = HIDDEN_DIM // NH
NPH = ND + NH

NC, NS, L = 2, 16, 16
NW = NC * NS


def _router_kernel(x_ref, c_ref, sel_ref, wts_ref, xbf_ref):
    xs = x_ref[...]
    cen = c_ref[...]
    xc = jnp.dot(xs, cen.T, preferred_element_type=jnp.float32,
                 precision=jax.lax.Precision.HIGHEST)
    xn = jnp.sum(xs * xs, axis=1, keepdims=True)
    cn = jnp.sum(cen * cen, axis=1)[None, :]
    dist = jnp.sqrt(jnp.maximum(xn - 2.0 * xc + cn, 0.0))
    m1 = jnp.max(dist, axis=1, keepdims=True)
    idx = jax.lax.broadcasted_iota(jnp.int32, dist.shape, 1)
    a1 = jnp.min(jnp.where(dist == m1, idx, NUM_EXPERTS), axis=1,
                 keepdims=True)
    masked = jnp.where(idx == a1, -jnp.inf, dist)
    m2 = jnp.max(masked, axis=1, keepdims=True)
    a2 = jnp.min(jnp.where(masked == m2, idx, NUM_EXPERTS), axis=1,
                 keepdims=True)
    z2 = jnp.exp(m2 - m1)
    w1 = 1.0 / (1.0 + z2)
    w2 = z2 / (1.0 + z2)
    sel_ref[...] = jnp.concatenate([a1, a2], axis=1)
    wts_ref[...] = jnp.concatenate([w1, w2], axis=1)
    xbf_ref[...] = xs.astype(jnp.bfloat16)


def _dispatch_body(sel_hbm, wts_hbm, srctok_hbm, gate_hbm, pos_hbm,
                   btab_hbm, sel_v, wts_v, srctok_v, gate_v, pos_v,
                   btab_v, cur_s):
    wid = lax.axis_index("s") * NC + lax.axis_index("c")

    @pl.when(wid == 0)
    def _():
        pltpu.sync_copy(sel_hbm, sel_v)
        pltpu.sync_copy(wts_hbm, wts_v)
        zi = jnp.zeros((L,), jnp.int32)
        zf = jnp.zeros((L,), jnp.float32)

        def _zero(i, _):
            srctok_v[pl.ds(i * L, L)] = zi
            gate_v[pl.ds(i * L, L)] = zf
            return _
        lax.fori_loop(0, NSLOTS // L, _zero, None)

        def _count(i, cnt):
            v = sel_v[pl.ds(i * L, L)]
            lanes = lax.iota(jnp.int32, L)
            for e in range(NUM_EXPERTS):
                ce = jnp.sum(jnp.where(v == e, 1, 0))
                cnt = cnt + jnp.where(lanes == e, ce, 0)
            return cnt
        cnt = lax.fori_loop(0, NASSIGN // L, _count, jnp.zeros((L,), jnp.int32))

        nblk = (cnt + (BT - 1)) // BT
        csum = plsc.cumsum(nblk)
        first_blk = csum - nblk
        seg_start = first_blk * BT
        total = jnp.sum(nblk)

        for e in range(NUM_EXPERTS):
            cur_s[e] = seg_start[e]

        ce_list = [csum[e] for e in range(NUM_EXPERTS)]
        last_e = jnp.int32(0)
        for ce in ce_list:
            last_e = last_e + jnp.where(ce <= total - 1, 1, 0)
        for c in range(NBPAD // L):
            bvec = lax.iota(jnp.int32, L) + c * L
            bexp = jnp.zeros((L,), jnp.int32)
            for ce in ce_list:
                bexp = bexp + jnp.where(bvec >= ce, 1, 0)
            valid = bvec < total
            bexp = jnp.where(valid, bexp, last_e)
            bfetch = jnp.where(valid, bvec, total - 1)
            btab_v[pl.ds(c * L, L)] = bexp
            btab_v[pl.ds(NBPAD + c * L, L)] = bfetch
            btab_v[pl.ds(2 * NBPAD + c * L, L)] = jnp.where(valid, 1, 0)

        def _scatter(i, _):
            v = sel_v[pl.ds(i * L, L)]
            w = wts_v[pl.ds(i * L, L)]
            tok = (lax.iota(jnp.int32, L) + i * L) // 2
            posv = jnp.zeros((L,), jnp.int32)
            for e in range(NUM_EXPERTS):
                m = v == e
                mi = jnp.where(m, 1, 0)
                rank = plsc.cumsum(mi) - 1
                base = cur_s[e]
                posv = jnp.where(m, base + rank, posv)
                cur_s[e] = base + jnp.sum(mi)
            plsc.store_scatter(srctok_v, [posv], tok)
            plsc.store_scatter(gate_v, [posv], w)
            pos_v[pl.ds(i * L, L)] = posv
            return _
        lax.fori_loop(0, NASSIGN // L, _scatter, None)

        pltpu.sync_copy(srctok_v, srctok_hbm)
        pltpu.sync_copy(gate_v, gate_hbm)
        pltpu.sync_copy(pos_v, pos_hbm)
        pltpu.sync_copy(btab_v, btab_hbm)


def _ffn_kernel(be_ref, bv_ref, bf_ref, x_ref, tok_ref, wg_ref, bg_ref,
                w1_ref, b1_ref, w2_ref, b2_ref, gate_ref, out_ref,
                g_acc, u_acc, xs_scr):
    b = pl.program_id(0)
    ph = pl.program_id(1)

    @pl.when(bv_ref[b] == 1)
    def _body():
        @pl.when(ph == 0)
        def _gather():
            tok = tok_ref[...]
            col = jax.lax.broadcasted_iota(jnp.int32, (BT, SEQ), 1)
            onehot = jnp.where(col == tok, 1.0, 0.0).astype(jnp.bfloat16)
            xs_scr[...] = jnp.dot(onehot, x_ref[...],
                                  preferred_element_type=jnp.float32)

        @pl.when(ph < ND)
        def _accum():
            xs = xs_scr[:, pl.ds(ph * DT, DT)]
            g = jnp.dot(xs, wg_ref[0], preferred_element_type=jnp.float32)
            u = jnp.dot(xs, w1_ref[0], preferred_element_type=jnp.float32)
            for jj in range(NH):
                gj = g[:, jj * HT:(jj + 1) * HT]
                uj = u[:, jj * HT:(jj + 1) * HT]

                @pl.when(ph == 0)
                def _(jj=jj, gj=gj, uj=uj):
                    g_acc[jj] = gj + bg_ref[0, 0][None, jj * HT:(jj + 1) * HT]
                    u_acc[jj] = uj + b1_ref[0, 0][None, jj * HT:(jj + 1) * HT]

                @pl.when(ph > 0)
                def _(jj=jj, gj=gj, uj=uj):
                    g_acc[jj] += gj
                    u_acc[jj] += uj

        @pl.when(ph >= ND)
        def _w2():
            j = ph - ND
            gate = gate_ref[...]
            g = g_acc[j]
            u = u_acc[j]
            hmid = (gate * (g * jax.nn.sigmoid(g))) * u
            y = jnp.dot(hmid, w2_ref[0], preferred_element_type=jnp.float32)

            @pl.when(j == 0)
            def _():
                out_ref[...] = y + gate * b2_ref[0, 0][None, :]

            @pl.when(j > 0)
            def _():
                out_ref[...] += y


def _ffn_in_specs():
    def _dclamp(p, bv_b):
        return jnp.where(bv_b == 1, jnp.minimum(p, ND - 1), ND - 1)

    return [
        pl.BlockSpec((SEQ, IN_DIM), lambda b, p, be, bv, bf: (0, 0)),
        pl.BlockSpec((BT, 1), lambda b, p, be, bv, bf: (bf[b], 0)),
        pl.BlockSpec((1, DT, HIDDEN_DIM),
                     lambda b, p, be, bv, bf: (be[b], _dclamp(p, bv[b]), 0)),
        pl.BlockSpec((1, 1, HIDDEN_DIM),
                     lambda b, p, be, bv, bf: (be[b], 0, 0)),
        pl.BlockSpec((1, DT, HIDDEN_DIM),
                     lambda b, p, be, bv, bf: (be[b], _dclamp(p, bv[b]), 0)),
        pl.BlockSpec((1, 1, HIDDEN_DIM),
                     lambda b, p, be, bv, bf: (be[b], 0, 0)),
        pl.BlockSpec((1, HT, IN_DIM),
                     lambda b, p, be, bv, bf:
                     (be[b], jnp.where(bv[b] == 1,
                                       jnp.clip(p - ND, 0, NH - 1),
                                       NH - 1), 0)),
        pl.BlockSpec((1, 1, IN_DIM),
                     lambda b, p, be, bv, bf: (be[b], 0, 0)),
        pl.BlockSpec((BT, 1), lambda b, p, be, bv, bf: (bf[b], 0)),
    ]


def _ffn_out_spec():
    return pl.BlockSpec((BT, IN_DIM), lambda b, p, be, bv, bf: (bf[b], 0))


def _ffn_scratch():
    return [pltpu.VMEM((NH, BT, HT), jnp.float32),
            pltpu.VMEM((NH, BT, HT), jnp.float32),
            pltpu.VMEM((BT, IN_DIM), jnp.float32)]


def _combine_body(y_hbm, pos_hbm, out_hbm, idx_v, buf_v, obuf_v, sem):
    wid = lax.axis_index("s") * NC + lax.axis_index("c")
    tok_per_w = SEQ // NW
    base_t = wid * tok_per_w
    pltpu.sync_copy(pos_hbm.at[pl.ds(base_t * 2, tok_per_w * 2)], idx_v)
    chunk = 32
    for c in range(tok_per_w // chunk):
        pltpu.async_copy(y_hbm.at[idx_v.at[pl.ds(c * chunk * 2, chunk * 2)]],
                         buf_v, sem).wait()

        def _comb(i, _):
            for j in range(IN_DIM // L):
                s = pl.ds(j * L, L)
                obuf_v[i, s] = buf_v[2 * i, s] + buf_v[2 * i + 1, s]
            return _
        lax.fori_loop(0, chunk, _comb, None)
        pltpu.sync_copy(obuf_v,
                        out_hbm.at[pl.ds(base_t + c * chunk, chunk)])


@jax.jit
def _moe_forward(xs, centroid, Wg, bg, W1, b1, W2, b2):
    _sc_mesh = plsc.VectorSubcoreMesh(core_axis_name="c", subcore_axis_name="s")
    sel, wts, x_bf = pl.pallas_call(
        _router_kernel,
        out_shape=[jax.ShapeDtypeStruct((SEQ, 2), jnp.int32),
                   jax.ShapeDtypeStruct((SEQ, 2), jnp.float32),
                   jax.ShapeDtypeStruct((SEQ, IN_DIM), jnp.bfloat16)],
    )(xs, centroid)

    dispatch = pl.kernel(
        _dispatch_body, mesh=_sc_mesh,
        out_type=[jax.ShapeDtypeStruct((NSLOTS,), jnp.int32),
                  jax.ShapeDtypeStruct((NSLOTS,), jnp.float32),
                  jax.ShapeDtypeStruct((NASSIGN,), jnp.int32),
                  jax.ShapeDtypeStruct((3 * NBPAD,), jnp.int32)],
        scratch_types=[pltpu.VMEM((NASSIGN,), jnp.int32),
                       pltpu.VMEM((NASSIGN,), jnp.float32),
                       pltpu.VMEM((NSLOTS,), jnp.int32),
                       pltpu.VMEM((NSLOTS,), jnp.float32),
                       pltpu.VMEM((NASSIGN,), jnp.int32),
                       pltpu.VMEM((3 * NBPAD,), jnp.int32),
                       pltpu.SMEM((NUM_EXPERTS,), jnp.int32)],
        compiler_params=pltpu.CompilerParams(needs_layout_passes=False),
    )
    srctok, slot_gate, pos, btab = dispatch(sel.reshape(NASSIGN),
                                            wts.reshape(NASSIGN))

    btab32 = btab.reshape(3, NBPAD)
    bexp, bfetch, bval = btab32[0], btab32[1], btab32[2]

    grid_spec = pltpu.PrefetchScalarGridSpec(
        num_scalar_prefetch=3,
        grid=(NBLOCKS, NPH),
        in_specs=_ffn_in_specs(),
        out_specs=_ffn_out_spec(),
        scratch_shapes=_ffn_scratch(),
    )
    y_sorted = pl.pallas_call(
        _ffn_kernel,
        grid_spec=grid_spec,
        out_shape=jax.ShapeDtypeStruct((NSLOTS, IN_DIM), jnp.float32),
        compiler_params=pltpu.CompilerParams(
            vmem_limit_bytes=64 * 1024 * 1024),
    )(bexp, bval, bfetch, x_bf, srctok.reshape(NSLOTS, 1), Wg,
      bg.reshape(NUM_EXPERTS, 1, HIDDEN_DIM), W1,
      b1.reshape(NUM_EXPERTS, 1, HIDDEN_DIM), W2,
      b2.reshape(NUM_EXPERTS, 1, IN_DIM), slot_gate.reshape(NSLOTS, 1))

    combine = pl.kernel(
        _combine_body, mesh=_sc_mesh,
        out_type=[jax.ShapeDtypeStruct((SEQ, IN_DIM), jnp.float32)],
        scratch_types=[pltpu.VMEM((2 * SEQ // NW,), jnp.int32),
                       pltpu.VMEM((64, IN_DIM), jnp.float32),
                       pltpu.VMEM((32, IN_DIM), jnp.float32),
                       pltpu.SemaphoreType.DMA],
        compiler_params=pltpu.CompilerParams(needs_layout_passes=False),
    )
    (out,) = combine(y_sorted, pos)
    return out


def kernel(x, centroid, Wg, bg, W1, b1, W2, b2):
    xs = x.reshape(-1, IN_DIM)
    out = _moe_forward(xs, centroid, Wg, bg, W1, b1, W2, b2)
    return out.reshape(x.shape)
